# Initial kernel scaffold; baseline (speedup 1.0000x reference)
import jax
import jax.numpy as jnp
from jax.experimental import pallas as pl


def kernel(x, edge_index, W0, A0, Wih0, bih0, bhh0, W1, A1, Wih1, bih1, bhh1, W2, A2, Wih2, bih2, bhh2):
    raise NotImplementedError("write your pallas kernel here")


# speedup vs baseline: 508.1851x; 508.1851x over previous
import jax
import jax.numpy as jnp
from jax import lax
from jax.experimental import pallas as pl
from jax.experimental.pallas import tpu as pltpu
from jax.experimental.pallas import tpu_sc as plsc

N = 10000
E = 320000
NPAD = 10240
LANES = 16
NC, NS = 2, 16
NW = NC * NS
EPW = E // NW
BLK = 1024
D0 = 192
NCLS = 16


def _sc_mark_body(dst_hbm, out_hbm, idx_v, mark_v):
    wid = lax.axis_index("s") * NC + lax.axis_index("c")
    pltpu.sync_copy(dst_hbm.at[pl.ds(wid * EPW, EPW)], idx_v)
    zeros = jnp.zeros((LANES,), jnp.float32)

    def zero_body(j, carry):
        mark_v[pl.ds(j * LANES, LANES)] = zeros
        return carry

    lax.fori_loop(0, NPAD // LANES, zero_body, 0)
    ones = jnp.ones((LANES,), jnp.float32)

    def scat_body(j, carry):
        idx16 = idx_v[pl.ds(j * LANES, LANES)]
        plsc.store_scatter(mark_v, [idx16], ones)
        return carry

    lax.fori_loop(0, EPW // LANES, scat_body, 0)
    pltpu.sync_copy(mark_v, out_hbm.at[pl.ds(wid * NPAD, NPAD)])


def _sc_marks(dst):
    mesh = plsc.VectorSubcoreMesh(core_axis_name="c", subcore_axis_name="s")
    return pl.kernel(
        _sc_mark_body,
        out_type=jax.ShapeDtypeStruct((NW * NPAD,), jnp.float32),
        mesh=mesh,
        compiler_params=pltpu.CompilerParams(needs_layout_passes=False),
        scratch_types=[
            pltpu.VMEM((EPW,), jnp.int32),
            pltpu.VMEM((NPAD,), jnp.float32),
        ],
    )(dst)


def _tc_dense_body(x_ref, mk_ref,
                   w0t, wi0, wg0, wo0, bi0, bg0, bo0,
                   w1t, wi1, wg1, wo1, bi1, bg1, bo1,
                   w2t, wi2, wg2, wo2, bi2, bg2, bo2,
                   out_ref):
    f32 = jnp.float32
    mask = (jnp.max(mk_ref[...], axis=1, keepdims=True) > 0.0).astype(f32)

    def lstm(u, wi, wg, wo, bi, bg, bo):
        gi = jax.nn.sigmoid(jnp.dot(u, wi[...], preferred_element_type=f32) + bi[...])
        gg = jnp.tanh(jnp.dot(u, wg[...], preferred_element_type=f32) + bg[...])
        go = jax.nn.sigmoid(jnp.dot(u, wo[...], preferred_element_type=f32) + bo[...])
        return go * jnp.tanh(gi * gg)

    xb = x_ref[...]
    u = jnp.maximum(jnp.dot(xb, w0t[...], preferred_element_type=f32), 0.0) * mask
    h = lstm(u, wi0, wg0, wo0, bi0, bg0, bo0)
    u = jnp.maximum(jnp.dot(h, w1t[...], preferred_element_type=f32), 0.0) * mask
    h = lstm(u, wi1, wg1, wo1, bi1, bg1, bo1)
    u = jnp.dot(h, w2t[...], preferred_element_type=f32) * mask
    out_ref[...] = lstm(u, wi2, wg2, wo2, bi2, bg2, bo2)


def _gate_params(Wih, bih, bhh, H):
    b = bih + bhh
    Wt = Wih.T
    return (Wt[:, 0:H], Wt[:, 2 * H:3 * H], Wt[:, 3 * H:4 * H],
            b[0:H].reshape(1, H), b[2 * H:3 * H].reshape(1, H), b[3 * H:4 * H].reshape(1, H))


def _tc_specs():
    full = lambda shape: pl.BlockSpec(shape, lambda i: (0, 0))
    lstm_specs = lambda H: [full((H, H))] * 3 + [full((1, H))] * 3
    return ([pl.BlockSpec((BLK, 128), lambda i: (i, 0)),
             pl.BlockSpec((BLK, NW), lambda i: (i, 0)),
             full((128, D0))] + lstm_specs(D0)
            + [full((D0, D0))] + lstm_specs(D0)
            + [full((D0, NCLS))] + lstm_specs(NCLS))


def kernel(x, edge_index, W0, A0, Wih0, bih0, bhh0, W1, A1, Wih1, bih1, bhh1,
           W2, A2, Wih2, bih2, bhh2):
    dst = edge_index[1].astype(jnp.int32)
    marks = _sc_marks(dst)
    mk = marks.reshape(NW, NPAD).T
    xp = jnp.pad(x, ((0, NPAD - N), (0, 0)))

    g0 = _gate_params(Wih0, bih0, bhh0, D0)
    g1 = _gate_params(Wih1, bih1, bhh1, D0)
    g2 = _gate_params(Wih2, bih2, bhh2, NCLS)
    w0t = W0.reshape(D0, 128).T
    w1t = W1.reshape(D0, D0).T
    w2t = W2.reshape(NCLS, D0).T

    out = pl.pallas_call(
        _tc_dense_body,
        grid=(NPAD // BLK,),
        in_specs=_tc_specs(),
        out_specs=pl.BlockSpec((BLK, NCLS), lambda i: (i, 0)),
        out_shape=jax.ShapeDtypeStruct((NPAD, NCLS), jnp.float32),
    )(xp, mk, w0t, *g0, w1t, *g1, w2t, *g2)
    return out[:N]
